# Initial kernel scaffold; baseline (speedup 1.0000x reference)
import functools

import jax
import jax.numpy as jnp
from jax import lax
from jax.experimental import pallas as pl
from jax.experimental.pallas import tpu as pltpu
from jax.experimental.pallas import tpu_sc as plsc

N = 10000
E = 160000
D = 256
HALF = 128
NPAD = 10240
N_CLASS = 10
FC_HIDDEN = 512

NTILES = 16
CHUNK = 128
ROWS_PER_TILE = NPAD // NTILES
E_TILE = -(-E // (NTILES * CHUNK)) * CHUNK
E_PAD = E_TILE * NTILES
NCHUNK = E_TILE // CHUNK


def _sc_agg_call(with_cnt):
    outs = [jax.ShapeDtypeStruct((2, NPAD, HALF), jnp.float32)]
    if with_cnt:
        outs.append(jax.ShapeDtypeStruct((NPAD, 16), jnp.float32))
    scratch = [
        pltpu.VMEM((CHUNK,), jnp.int32),
        pltpu.VMEM((CHUNK,), jnp.int32),
        pltpu.VMEM((CHUNK, HALF), jnp.float32),
        pltpu.VMEM((CHUNK, HALF), jnp.float32),
        pltpu.VMEM_SHARED((NPAD, HALF), jnp.float32),
        pltpu.SemaphoreType.DMA,
    ]
    if with_cnt:
        scratch += [
            pltpu.VMEM((CHUNK, 16), jnp.float32),
            pltpu.VMEM((CHUNK, 16), jnp.float32),
            pltpu.VMEM_SHARED((NPAD, 16), jnp.float32),
        ]

    def body(h_hbm, srcb_hbm, dst_hbm, agg_hbm, *rest):
        if with_cnt:
            cnt_hbm, src_v, dst_v, rows_v, zero_v, agg_s, sem, ones_v, z16_v, cnt_s = rest
        else:
            src_v, dst_v, rows_v, zero_v, agg_s, sem = rest
        c = lax.axis_index("c")
        s = lax.axis_index("s")

        @pl.loop(0, CHUNK)
        def _init(i):
            for j in range(HALF // 16):
                zero_v[i, pl.ds(j * 16, 16)] = jnp.zeros((16,), jnp.float32)
            if with_cnt:
                ones_v[i, :] = jnp.ones((16,), jnp.float32)
                z16_v[i, :] = jnp.zeros((16,), jnp.float32)

        @pl.loop(0, ROWS_PER_TILE // CHUNK)
        def _zero(j):
            base = s * ROWS_PER_TILE + j * CHUNK
            pltpu.sync_copy(zero_v, agg_s.at[pl.ds(base, CHUNK)])
            if with_cnt:
                pltpu.sync_copy(z16_v, cnt_s.at[pl.ds(base, CHUNK)])

        plsc.subcore_barrier()

        ebase = s * E_TILE

        @pl.loop(0, NCHUNK)
        def _acc(j):
            off = ebase + j * CHUNK
            pltpu.sync_copy(srcb_hbm.at[c, pl.ds(off, CHUNK)], src_v)
            pltpu.sync_copy(dst_hbm.at[pl.ds(off, CHUNK)], dst_v)
            pltpu.async_copy(h_hbm.at[src_v], rows_v, sem).wait()
            pltpu.sync_copy(rows_v, agg_s.at[dst_v], add=True)
            if with_cnt:
                pltpu.sync_copy(ones_v, cnt_s.at[dst_v], add=True)

        plsc.subcore_barrier()

        r0 = s * ROWS_PER_TILE
        pltpu.sync_copy(agg_s.at[pl.ds(r0, ROWS_PER_TILE)],
                        agg_hbm.at[c, pl.ds(r0, ROWS_PER_TILE)])
        if with_cnt:
            @pl.when(c == 0)
            def _flush_cnt():
                pltpu.sync_copy(cnt_s.at[pl.ds(r0, ROWS_PER_TILE)],
                                cnt_hbm.at[pl.ds(r0, ROWS_PER_TILE)])

    mesh = plsc.VectorSubcoreMesh(core_axis_name="c", subcore_axis_name="s")
    return pl.kernel(body, out_type=tuple(outs) if with_cnt else outs[0],
                     mesh=mesh, scratch_types=scratch)


_sc_agg_cnt = _sc_agg_call(True)
_sc_agg = _sc_agg_call(False)


_BM = 512


def _tc_layer_body(agg_a, agg_b, h_a, h_b, cnt, wl, wr, bl, out):
    o = pl.program_id(0)
    rb = pl.program_id(1)
    rec = 1.0 / jnp.maximum(cnt[...], 1.0)
    ma = agg_a[0] * rec
    mb = agg_b[0] * rec
    acc = jnp.dot(ma, wl[0:HALF, :], preferred_element_type=jnp.float32)
    acc += jnp.dot(mb, wl[HALF:D, :], preferred_element_type=jnp.float32)
    acc += jnp.dot(h_a[0], wr[0:HALF, :], preferred_element_type=jnp.float32)
    acc += jnp.dot(h_b[0], wr[HALF:D, :], preferred_element_type=jnp.float32)
    acc += bl[...]
    acc = jnp.maximum(acc, 0.0)
    row = rb * _BM + lax.broadcasted_iota(jnp.int32, (_BM, HALF), 0)
    out[0] = jnp.where(row < N, acc, 0.0)


def _tc_layer(agg, h, cnt, wl, wr, bl):
    nrb = NPAD // _BM
    grid = (2, nrb)
    specs = [
        pl.BlockSpec((1, _BM, HALF), lambda o, rb: (0, rb, 0)),
        pl.BlockSpec((1, _BM, HALF), lambda o, rb: (1, rb, 0)),
        pl.BlockSpec((1, _BM, HALF), lambda o, rb: (0, rb, 0)),
        pl.BlockSpec((1, _BM, HALF), lambda o, rb: (1, rb, 0)),
        pl.BlockSpec((_BM, 1), lambda o, rb: (rb, 0)),
        pl.BlockSpec((D, HALF), lambda o, rb: (0, o)),
        pl.BlockSpec((D, HALF), lambda o, rb: (0, o)),
        pl.BlockSpec((1, HALF), lambda o, rb: (0, o)),
    ]
    out_spec = pl.BlockSpec((1, _BM, HALF), lambda o, rb: (o, rb, 0))
    return pl.pallas_call(
        _tc_layer_body,
        grid=grid,
        in_specs=specs,
        out_specs=out_spec,
        out_shape=jax.ShapeDtypeStruct((2, NPAD, HALF), jnp.float32),
    )(agg, agg, h, h, cnt, wl, wr, bl)


def _head_body(h1, h2, h3, w1, b1, w2, b2, out, a1, a2, a3):
    rb = pl.program_id(0)
    nrb = pl.num_programs(0)

    @pl.when(rb == 0)
    def _():
        a1[...] = jnp.zeros_like(a1)
        a2[...] = jnp.zeros_like(a2)
        a3[...] = jnp.zeros_like(a3)

    a1[...] += jnp.sum(h1[...], axis=1)
    a2[...] += jnp.sum(h2[...], axis=1)
    a3[...] += jnp.sum(h3[...], axis=1)

    @pl.when(rb == nrb - 1)
    def _():
        cat = jnp.concatenate(
            [a1[0:1, :], a1[1:2, :], a2[0:1, :], a2[1:2, :],
             a3[0:1, :], a3[1:2, :]], axis=1)
        pooled = cat * (1.0 / N)
        z = jnp.maximum(
            jnp.dot(pooled, w1[...], preferred_element_type=jnp.float32)
            + b1[...], 0.0)
        out[...] = (jnp.dot(z, w2[...], preferred_element_type=jnp.float32)
                    + b2[...])


def _head(h1, h2, h3, w1, b1, w2, b2):
    nrb = NPAD // _BM
    hspec = pl.BlockSpec((2, _BM, HALF), lambda rb: (0, rb, 0))
    return pl.pallas_call(
        _head_body,
        grid=(nrb,),
        in_specs=[
            hspec, hspec, hspec,
            pl.BlockSpec((D * 3, FC_HIDDEN), lambda rb: (0, 0)),
            pl.BlockSpec((1, FC_HIDDEN), lambda rb: (0, 0)),
            pl.BlockSpec((FC_HIDDEN, N_CLASS), lambda rb: (0, 0)),
            pl.BlockSpec((1, N_CLASS), lambda rb: (0, 0)),
        ],
        out_specs=pl.BlockSpec((1, N_CLASS), lambda rb: (0, 0)),
        out_shape=jax.ShapeDtypeStruct((1, N_CLASS), jnp.float32),
        scratch_shapes=[pltpu.VMEM((2, HALF), jnp.float32)] * 3,
    )(h1, h2, h3, w1, b1, w2, b2)


def kernel(x, edge_index, Wl0, bl0, Wr0, Wl1, bl1, Wr1, Wl2, bl2, Wr2,
           fc1_W, fc1_b, fc2_W, fc2_b):
    src = edge_index[0]
    dst = edge_index[1]
    pad = E_PAD - E
    src_p = jnp.concatenate([src, jnp.zeros((pad,), jnp.int32)])
    dst_p = jnp.concatenate([dst, jnp.full((pad,), N, jnp.int32)])
    src_both = jnp.stack([src_p, src_p + NPAD])

    xp = jnp.pad(x, ((0, NPAD - N), (0, 0)))
    h = jnp.transpose(xp.reshape(NPAD, 2, HALF), (1, 0, 2))

    agg, cnt16 = _sc_agg_cnt(h.reshape(2 * NPAD, HALF), src_both, dst_p)
    cnt = cnt16[:, 0:1]
    h1 = _tc_layer(agg, h, cnt, Wl0, Wr0, bl0.reshape(1, D))

    agg = _sc_agg(h1.reshape(2 * NPAD, HALF), src_both, dst_p)
    h2 = _tc_layer(agg, h1, cnt, Wl1, Wr1, bl1.reshape(1, D))
    agg = _sc_agg(h2.reshape(2 * NPAD, HALF), src_both, dst_p)
    h3 = _tc_layer(agg, h2, cnt, Wl2, Wr2, bl2.reshape(1, D))

    return _head(h1, h2, h3, fc1_W, fc1_b.reshape(1, FC_HIDDEN),
                 fc2_W, fc2_b.reshape(1, N_CLASS))


# speedup vs baseline: 2.6216x; 2.6216x over previous
import functools

import jax
import jax.numpy as jnp
from jax import lax
from jax.experimental import pallas as pl
from jax.experimental.pallas import tpu as pltpu
from jax.experimental.pallas import tpu_sc as plsc

N = 10000
E = 160000
D = 256
HALF = 128
NPAD = 10240
N_CLASS = 10
FC_HIDDEN = 512

NTILES = 16
CHUNK = 64
ROWS_PER_TILE = NPAD // NTILES
E_TILE = -(-E // (NTILES * CHUNK)) * CHUNK
E_PAD = E_TILE * NTILES
NCHUNK = E_TILE // CHUNK


def _make_sc_agg():
    scratch = [
        pltpu.VMEM((CHUNK,), jnp.int32),
        pltpu.VMEM((CHUNK,), jnp.int32),
        pltpu.VMEM((CHUNK, HALF), jnp.float32),
        pltpu.VMEM_SHARED((NPAD, HALF), jnp.float32),
        pltpu.SemaphoreType.DMA,
    ]

    def body(h_hbm, srcb_hbm, dst_hbm, agg_hbm, src_v, dst_v, rows_v, agg_s,
             sem):
        c = lax.axis_index("c")
        s = lax.axis_index("s")

        @pl.loop(0, CHUNK)
        def _init(i):
            for j in range(HALF // 16):
                rows_v[i, pl.ds(j * 16, 16)] = jnp.zeros((16,), jnp.float32)

        @pl.loop(0, ROWS_PER_TILE // CHUNK)
        def _zero(j):
            base = s * ROWS_PER_TILE + j * CHUNK
            pltpu.sync_copy(rows_v, agg_s.at[pl.ds(base, CHUNK)])

        plsc.subcore_barrier()

        ebase = s * E_TILE

        @pl.loop(0, NCHUNK)
        def _acc(j):
            off = ebase + j * CHUNK
            pltpu.sync_copy(srcb_hbm.at[c, pl.ds(off, CHUNK)], src_v)
            pltpu.sync_copy(dst_hbm.at[pl.ds(off, CHUNK)], dst_v)
            pltpu.async_copy(h_hbm.at[src_v], rows_v, sem).wait()
            pltpu.sync_copy(rows_v, agg_s.at[dst_v], add=True)

        plsc.subcore_barrier()

        @pl.loop(0, ROWS_PER_TILE // CHUNK)
        def _flush(j):
            base = s * ROWS_PER_TILE + j * CHUNK
            pltpu.sync_copy(agg_s.at[pl.ds(base, CHUNK)], rows_v)
            pltpu.sync_copy(rows_v, agg_hbm.at[c, pl.ds(base, CHUNK)])

    mesh = plsc.VectorSubcoreMesh(core_axis_name="c", subcore_axis_name="s")
    return pl.kernel(body,
                     out_type=jax.ShapeDtypeStruct((2, NPAD, HALF),
                                                   jnp.float32),
                     mesh=mesh, scratch_types=scratch)


def _make_sc_cnt():
    scratch = [
        pltpu.VMEM((CHUNK,), jnp.int32),
        pltpu.VMEM((CHUNK, HALF), jnp.float32),
        pltpu.VMEM_SHARED((NPAD, HALF), jnp.float32),
    ]

    def body(dst_hbm, cnt_hbm, dst_v, ones_v, cnt_s):
        c = lax.axis_index("c")
        s = lax.axis_index("s")

        @pl.loop(0, CHUNK)
        def _init(i):
            for j in range(HALF // 16):
                ones_v[i, pl.ds(j * 16, 16)] = jnp.zeros((16,), jnp.float32)

        @pl.loop(0, ROWS_PER_TILE // CHUNK)
        def _zero(j):
            base = s * ROWS_PER_TILE + j * CHUNK
            pltpu.sync_copy(ones_v, cnt_s.at[pl.ds(base, CHUNK)])

        @pl.loop(0, CHUNK)
        def _setones(i):
            for j in range(HALF // 16):
                ones_v[i, pl.ds(j * 16, 16)] = jnp.ones((16,), jnp.float32)

        plsc.subcore_barrier()

        ebase = s * E_TILE

        @pl.loop(0, NCHUNK)
        def _acc(j):
            off = ebase + j * CHUNK
            pltpu.sync_copy(dst_hbm.at[pl.ds(off, CHUNK)], dst_v)
            pltpu.sync_copy(ones_v, cnt_s.at[dst_v], add=True)

        plsc.subcore_barrier()

        @pl.loop(0, ROWS_PER_TILE // CHUNK)
        def _flush(j):
            base = s * ROWS_PER_TILE + j * CHUNK

            @pl.when(c == 0)
            def _f():
                pltpu.sync_copy(cnt_s.at[pl.ds(base, CHUNK)], ones_v)
                pltpu.sync_copy(ones_v, cnt_hbm.at[pl.ds(base, CHUNK)])

    mesh = plsc.VectorSubcoreMesh(core_axis_name="c", subcore_axis_name="s")
    return pl.kernel(body,
                     out_type=jax.ShapeDtypeStruct((NPAD, HALF), jnp.float32),
                     mesh=mesh, scratch_types=scratch)


_sc_agg = _make_sc_agg()
_sc_cnt = _make_sc_cnt()


_BM = 512


def _tc_layer_body(agg_a, agg_b, h_a, h_b, cnt, wl, wr, bl, out):
    o = pl.program_id(0)
    rb = pl.program_id(1)
    rec = 1.0 / jnp.maximum(cnt[...], 1.0)
    ma = agg_a[0] * rec
    mb = agg_b[0] * rec
    acc = jnp.dot(ma, wl[0:HALF, :], preferred_element_type=jnp.float32)
    acc += jnp.dot(mb, wl[HALF:D, :], preferred_element_type=jnp.float32)
    acc += jnp.dot(h_a[0], wr[0:HALF, :], preferred_element_type=jnp.float32)
    acc += jnp.dot(h_b[0], wr[HALF:D, :], preferred_element_type=jnp.float32)
    acc += bl[...]
    acc = jnp.maximum(acc, 0.0)
    row = rb * _BM + lax.broadcasted_iota(jnp.int32, (_BM, HALF), 0)
    out[0] = jnp.where(row < N, acc, 0.0)


def _tc_layer(agg, h, cnt, wl, wr, bl):
    nrb = NPAD // _BM
    grid = (2, nrb)
    specs = [
        pl.BlockSpec((1, _BM, HALF), lambda o, rb: (0, rb, 0)),
        pl.BlockSpec((1, _BM, HALF), lambda o, rb: (1, rb, 0)),
        pl.BlockSpec((1, _BM, HALF), lambda o, rb: (0, rb, 0)),
        pl.BlockSpec((1, _BM, HALF), lambda o, rb: (1, rb, 0)),
        pl.BlockSpec((_BM, 1), lambda o, rb: (rb, 0)),
        pl.BlockSpec((D, HALF), lambda o, rb: (0, o)),
        pl.BlockSpec((D, HALF), lambda o, rb: (0, o)),
        pl.BlockSpec((1, HALF), lambda o, rb: (0, o)),
    ]
    out_spec = pl.BlockSpec((1, _BM, HALF), lambda o, rb: (o, rb, 0))
    return pl.pallas_call(
        _tc_layer_body,
        grid=grid,
        in_specs=specs,
        out_specs=out_spec,
        out_shape=jax.ShapeDtypeStruct((2, NPAD, HALF), jnp.float32),
    )(agg, agg, h, h, cnt, wl, wr, bl)


def _head_body(h1, h2, h3, w1, b1, w2, b2, out, a1, a2, a3):
    rb = pl.program_id(0)
    nrb = pl.num_programs(0)

    @pl.when(rb == 0)
    def _():
        a1[...] = jnp.zeros_like(a1)
        a2[...] = jnp.zeros_like(a2)
        a3[...] = jnp.zeros_like(a3)

    a1[...] += jnp.sum(h1[...], axis=1)
    a2[...] += jnp.sum(h2[...], axis=1)
    a3[...] += jnp.sum(h3[...], axis=1)

    @pl.when(rb == nrb - 1)
    def _():
        cat = jnp.concatenate(
            [a1[0:1, :], a1[1:2, :], a2[0:1, :], a2[1:2, :],
             a3[0:1, :], a3[1:2, :]], axis=1)
        pooled = cat * (1.0 / N)
        z = jnp.maximum(
            jnp.dot(pooled, w1[...], preferred_element_type=jnp.float32)
            + b1[...], 0.0)
        out[...] = (jnp.dot(z, w2[...], preferred_element_type=jnp.float32)
                    + b2[...])


def _head(h1, h2, h3, w1, b1, w2, b2):
    nrb = NPAD // _BM
    hspec = pl.BlockSpec((2, _BM, HALF), lambda rb: (0, rb, 0))
    return pl.pallas_call(
        _head_body,
        grid=(nrb,),
        in_specs=[
            hspec, hspec, hspec,
            pl.BlockSpec((D * 3, FC_HIDDEN), lambda rb: (0, 0)),
            pl.BlockSpec((1, FC_HIDDEN), lambda rb: (0, 0)),
            pl.BlockSpec((FC_HIDDEN, N_CLASS), lambda rb: (0, 0)),
            pl.BlockSpec((1, N_CLASS), lambda rb: (0, 0)),
        ],
        out_specs=pl.BlockSpec((1, N_CLASS), lambda rb: (0, 0)),
        out_shape=jax.ShapeDtypeStruct((1, N_CLASS), jnp.float32),
        scratch_shapes=[pltpu.VMEM((2, HALF), jnp.float32)] * 3,
    )(h1, h2, h3, w1, b1, w2, b2)


def kernel(x, edge_index, Wl0, bl0, Wr0, Wl1, bl1, Wr1, Wl2, bl2, Wr2,
           fc1_W, fc1_b, fc2_W, fc2_b):
    src = edge_index[0]
    dst = edge_index[1]
    pad = E_PAD - E
    src_p = jnp.concatenate([src, jnp.zeros((pad,), jnp.int32)])
    dst_p = jnp.concatenate([dst, jnp.full((pad,), N, jnp.int32)])
    src_both = jnp.stack([src_p, src_p + NPAD])

    xp = jnp.pad(x, ((0, NPAD - N), (0, 0)))
    h = jnp.transpose(xp.reshape(NPAD, 2, HALF), (1, 0, 2))

    cnt = _sc_cnt(dst_p)[:, 0:1]

    agg = _sc_agg(h.reshape(2 * NPAD, HALF), src_both, dst_p)
    h1 = _tc_layer(agg, h, cnt, Wl0, Wr0, bl0.reshape(1, D))

    agg = _sc_agg(h1.reshape(2 * NPAD, HALF), src_both, dst_p)
    h2 = _tc_layer(agg, h1, cnt, Wl1, Wr1, bl1.reshape(1, D))
    agg = _sc_agg(h2.reshape(2 * NPAD, HALF), src_both, dst_p)
    h3 = _tc_layer(agg, h2, cnt, Wl2, Wr2, bl2.reshape(1, D))

    return _head(h1, h2, h3, fc1_W, fc1_b.reshape(1, FC_HIDDEN),
                 fc2_W, fc2_b.reshape(1, N_CLASS))


# speedup vs baseline: 3.1363x; 1.1963x over previous
import functools

import jax
import jax.numpy as jnp
from jax import lax
from jax.experimental import pallas as pl
from jax.experimental.pallas import tpu as pltpu
from jax.experimental.pallas import tpu_sc as plsc

N = 10000
E = 160000
D = 256
HALF = 128
NPAD = 10240
N_CLASS = 10
FC_HIDDEN = 512

NTILES = 16
CHUNK = 64
NB = 16
ROWS_PER_TILE = NPAD // NTILES
NCHUNK = -(-E // (NTILES * CHUNK * NB)) * NB
E_TILE = NCHUNK * CHUNK
E_PAD = E_TILE * NTILES
NBATCH = NCHUNK // NB
NCROWS = E_PAD // CHUNK


def _make_sc_agg():
    scratch = [
        pltpu.VMEM((NB, CHUNK), jnp.int32),
        pltpu.VMEM((NB, CHUNK), jnp.int32),
        pltpu.VMEM((2, CHUNK, HALF), jnp.float32),
        pltpu.VMEM_SHARED((NPAD, HALF), jnp.float32),
        pltpu.SemaphoreType.DMA,
        pltpu.SemaphoreType.DMA,
    ]

    def body(h_hbm, srcb_hbm, dst_hbm, agg_hbm, srcb_v, dstb_v, rows_v,
             agg_s, sem0, sem1):
        c = lax.axis_index("c")
        s = lax.axis_index("s")

        @pl.loop(0, CHUNK)
        def _init(i):
            for j in range(HALF // 16):
                rows_v[0, i, pl.ds(j * 16, 16)] = jnp.zeros((16,), jnp.float32)

        @pl.loop(0, ROWS_PER_TILE // CHUNK)
        def _zero(j):
            base = s * ROWS_PER_TILE + j * CHUNK
            pltpu.sync_copy(rows_v.at[0], agg_s.at[pl.ds(base, CHUNK)])

        plsc.subcore_barrier()

        crow0 = s * NCHUNK
        sems = (sem0, sem1)

        @pl.loop(0, NBATCH)
        def _batch(b):
            row0 = crow0 + b * NB
            pltpu.sync_copy(srcb_hbm.at[c, pl.ds(row0, NB)], srcb_v)
            pltpu.sync_copy(dst_hbm.at[pl.ds(row0, NB)], dstb_v)
            cps = [None, None]
            cps[0] = pltpu.async_copy(h_hbm.at[srcb_v.at[0]], rows_v.at[0],
                                      sems[0])
            for j in range(NB):
                if j + 1 < NB:
                    cps[(j + 1) % 2] = pltpu.async_copy(
                        h_hbm.at[srcb_v.at[j + 1]], rows_v.at[(j + 1) % 2],
                        sems[(j + 1) % 2])
                cps[j % 2].wait()
                pltpu.sync_copy(rows_v.at[j % 2], agg_s.at[dstb_v.at[j]],
                                add=True)

        plsc.subcore_barrier()

        @pl.loop(0, ROWS_PER_TILE // CHUNK)
        def _flush(j):
            base = s * ROWS_PER_TILE + j * CHUNK
            pltpu.sync_copy(agg_s.at[pl.ds(base, CHUNK)], rows_v.at[0])
            pltpu.sync_copy(rows_v.at[0], agg_hbm.at[c, pl.ds(base, CHUNK)])

    mesh = plsc.VectorSubcoreMesh(core_axis_name="c", subcore_axis_name="s")
    return pl.kernel(body,
                     out_type=jax.ShapeDtypeStruct((2, NPAD, HALF),
                                                   jnp.float32),
                     mesh=mesh, scratch_types=scratch)


def _make_sc_cnt():
    scratch = [
        pltpu.VMEM((NB, CHUNK), jnp.int32),
        pltpu.VMEM((CHUNK, HALF), jnp.float32),
        pltpu.VMEM_SHARED((NPAD, HALF), jnp.float32),
        pltpu.SemaphoreType.DMA,
    ]

    def body(dst_hbm, cnt_hbm, dstb_v, ones_v, cnt_s, sem):
        c = lax.axis_index("c")
        s = lax.axis_index("s")

        @pl.loop(0, CHUNK)
        def _init(i):
            for j in range(HALF // 16):
                ones_v[i, pl.ds(j * 16, 16)] = jnp.zeros((16,), jnp.float32)

        @pl.loop(0, ROWS_PER_TILE // CHUNK)
        def _zero(j):
            base = s * ROWS_PER_TILE + j * CHUNK
            pltpu.sync_copy(ones_v, cnt_s.at[pl.ds(base, CHUNK)])

        @pl.loop(0, CHUNK)
        def _setones(i):
            for j in range(HALF // 16):
                ones_v[i, pl.ds(j * 16, 16)] = jnp.ones((16,), jnp.float32)

        plsc.subcore_barrier()

        crow0 = s * NCHUNK

        @pl.loop(0, NBATCH)
        def _batch(b):
            row0 = crow0 + b * NB
            pltpu.sync_copy(dst_hbm.at[pl.ds(row0, NB)], dstb_v)
            cps = [pltpu.async_copy(ones_v, cnt_s.at[dstb_v.at[j]], sem,
                                    add=True) for j in range(NB)]
            for cp in cps:
                cp.wait()

        plsc.subcore_barrier()

        @pl.loop(0, ROWS_PER_TILE // CHUNK)
        def _flush(j):
            base = s * ROWS_PER_TILE + j * CHUNK

            @pl.when(c == 0)
            def _f():
                pltpu.sync_copy(cnt_s.at[pl.ds(base, CHUNK)], ones_v)
                pltpu.sync_copy(ones_v, cnt_hbm.at[pl.ds(base, CHUNK)])

    mesh = plsc.VectorSubcoreMesh(core_axis_name="c", subcore_axis_name="s")
    return pl.kernel(body,
                     out_type=jax.ShapeDtypeStruct((NPAD, HALF), jnp.float32),
                     mesh=mesh, scratch_types=scratch)


_sc_agg = _make_sc_agg()
_sc_cnt = _make_sc_cnt()


_BM = 512


def _tc_layer_body(agg_a, agg_b, h_a, h_b, cnt, wl, wr, bl, out):
    o = pl.program_id(0)
    rb = pl.program_id(1)
    rec = 1.0 / jnp.maximum(cnt[...], 1.0)
    ma = agg_a[0] * rec
    mb = agg_b[0] * rec
    acc = jnp.dot(ma, wl[0:HALF, :], preferred_element_type=jnp.float32)
    acc += jnp.dot(mb, wl[HALF:D, :], preferred_element_type=jnp.float32)
    acc += jnp.dot(h_a[0], wr[0:HALF, :], preferred_element_type=jnp.float32)
    acc += jnp.dot(h_b[0], wr[HALF:D, :], preferred_element_type=jnp.float32)
    acc += bl[...]
    acc = jnp.maximum(acc, 0.0)
    row = rb * _BM + lax.broadcasted_iota(jnp.int32, (_BM, HALF), 0)
    out[0] = jnp.where(row < N, acc, 0.0)


def _tc_layer(agg, h, cnt, wl, wr, bl):
    nrb = NPAD // _BM
    grid = (2, nrb)
    specs = [
        pl.BlockSpec((1, _BM, HALF), lambda o, rb: (0, rb, 0)),
        pl.BlockSpec((1, _BM, HALF), lambda o, rb: (1, rb, 0)),
        pl.BlockSpec((1, _BM, HALF), lambda o, rb: (0, rb, 0)),
        pl.BlockSpec((1, _BM, HALF), lambda o, rb: (1, rb, 0)),
        pl.BlockSpec((_BM, 1), lambda o, rb: (rb, 0)),
        pl.BlockSpec((D, HALF), lambda o, rb: (0, o)),
        pl.BlockSpec((D, HALF), lambda o, rb: (0, o)),
        pl.BlockSpec((1, HALF), lambda o, rb: (0, o)),
    ]
    out_spec = pl.BlockSpec((1, _BM, HALF), lambda o, rb: (o, rb, 0))
    return pl.pallas_call(
        _tc_layer_body,
        grid=grid,
        in_specs=specs,
        out_specs=out_spec,
        out_shape=jax.ShapeDtypeStruct((2, NPAD, HALF), jnp.float32),
    )(agg, agg, h, h, cnt, wl, wr, bl)


def _head_body(h1, h2, h3, w1, b1, w2, b2, out, a1, a2, a3):
    rb = pl.program_id(0)
    nrb = pl.num_programs(0)

    @pl.when(rb == 0)
    def _():
        a1[...] = jnp.zeros_like(a1)
        a2[...] = jnp.zeros_like(a2)
        a3[...] = jnp.zeros_like(a3)

    a1[...] += jnp.sum(h1[...], axis=1)
    a2[...] += jnp.sum(h2[...], axis=1)
    a3[...] += jnp.sum(h3[...], axis=1)

    @pl.when(rb == nrb - 1)
    def _():
        cat = jnp.concatenate(
            [a1[0:1, :], a1[1:2, :], a2[0:1, :], a2[1:2, :],
             a3[0:1, :], a3[1:2, :]], axis=1)
        pooled = cat * (1.0 / N)
        z = jnp.maximum(
            jnp.dot(pooled, w1[...], preferred_element_type=jnp.float32)
            + b1[...], 0.0)
        out[...] = (jnp.dot(z, w2[...], preferred_element_type=jnp.float32)
                    + b2[...])


def _head(h1, h2, h3, w1, b1, w2, b2):
    nrb = NPAD // _BM
    hspec = pl.BlockSpec((2, _BM, HALF), lambda rb: (0, rb, 0))
    return pl.pallas_call(
        _head_body,
        grid=(nrb,),
        in_specs=[
            hspec, hspec, hspec,
            pl.BlockSpec((D * 3, FC_HIDDEN), lambda rb: (0, 0)),
            pl.BlockSpec((1, FC_HIDDEN), lambda rb: (0, 0)),
            pl.BlockSpec((FC_HIDDEN, N_CLASS), lambda rb: (0, 0)),
            pl.BlockSpec((1, N_CLASS), lambda rb: (0, 0)),
        ],
        out_specs=pl.BlockSpec((1, N_CLASS), lambda rb: (0, 0)),
        out_shape=jax.ShapeDtypeStruct((1, N_CLASS), jnp.float32),
        scratch_shapes=[pltpu.VMEM((2, HALF), jnp.float32)] * 3,
    )(h1, h2, h3, w1, b1, w2, b2)


def kernel(x, edge_index, Wl0, bl0, Wr0, Wl1, bl1, Wr1, Wl2, bl2, Wr2,
           fc1_W, fc1_b, fc2_W, fc2_b):
    src = edge_index[0]
    dst = edge_index[1]
    pad = E_PAD - E
    src_p = jnp.concatenate([src, jnp.zeros((pad,), jnp.int32)])
    dst_p = jnp.concatenate([dst, jnp.full((pad,), N, jnp.int32)])
    src_both = jnp.stack([src_p, src_p + NPAD]).reshape(2, NCROWS, CHUNK)
    dst_p = dst_p.reshape(NCROWS, CHUNK)

    xp = jnp.pad(x, ((0, NPAD - N), (0, 0)))
    h = jnp.transpose(xp.reshape(NPAD, 2, HALF), (1, 0, 2))

    cnt = _sc_cnt(dst_p)[:, 0:1]

    agg = _sc_agg(h.reshape(2 * NPAD, HALF), src_both, dst_p)
    h1 = _tc_layer(agg, h, cnt, Wl0, Wr0, bl0.reshape(1, D))

    agg = _sc_agg(h1.reshape(2 * NPAD, HALF), src_both, dst_p)
    h2 = _tc_layer(agg, h1, cnt, Wl1, Wr1, bl1.reshape(1, D))
    agg = _sc_agg(h2.reshape(2 * NPAD, HALF), src_both, dst_p)
    h3 = _tc_layer(agg, h2, cnt, Wl2, Wr2, bl2.reshape(1, D))

    return _head(h1, h2, h3, fc1_W, fc1_b.reshape(1, FC_HIDDEN),
                 fc2_W, fc2_b.reshape(1, N_CLASS))
